# baseline (speedup 1.0000x reference)
import functools

import jax
import jax.numpy as jnp
from jax import lax
from jax.experimental import pallas as pl
from jax.experimental.pallas import tpu as pltpu
from jax.experimental.pallas import tpu_sc as plsc

WIN = 128
KAGG = 3
NSUB = 16
RB = 8192


def _ceil_to(x, m):
    return (x + m - 1) // m * m


_SC_PARAMS = pltpu.CompilerParams(use_tc_tiling_on_sc=False)


def _sc_aggregate(gl, gr, srcp2, dstp2, npad, epw):
    rows_per_sub = npad // NSUB
    wrows = epw // WIN
    pairs = wrows // (2 * KAGG)
    SW = KAGG * WIN

    @functools.partial(
        pl.kernel,
        out_type=[jax.ShapeDtypeStruct((npad, 32), jnp.float32)] * 2,
        mesh=plsc.VectorSubcoreMesh(core_axis_name="c", subcore_axis_name="s"),
        scratch_types=[
            pltpu.VMEM((KAGG, WIN), jnp.int32),
            pltpu.VMEM((KAGG, WIN), jnp.int32),
            pltpu.VMEM((KAGG, WIN), jnp.int32),
            pltpu.VMEM((KAGG, WIN), jnp.int32),
            pltpu.VMEM((SW, 32), jnp.float32),
            pltpu.VMEM((SW, 32), jnp.float32),
            pltpu.VMEM_SHARED((npad, 32), jnp.float32),
            pltpu.SemaphoreType.DMA,
            pltpu.SemaphoreType.DMA,
            pltpu.SemaphoreType.DMA,
            pltpu.SemaphoreType.DMA,
            pltpu.SemaphoreType.DMA,
            pltpu.SemaphoreType.DMA,
        ],
        compiler_params=_SC_PARAMS,
    )
    def k(gl_hbm, gr_hbm, src_hbm, dst_hbm, ol_hbm, or_hbm,
          sidxa, didxa, sidxb, didxb, rowsa, rowsb, acc,
          semia, semib, semga, semgb, semsa, semsb):
        c = lax.axis_index("c")
        s = lax.axis_index("s")
        rbase = pl.multiple_of(s * rows_per_sub, 8)
        wbase = pl.multiple_of(s * wrows, 8)

        def run(g_hbm, o_hbm):
            def idx_fetch(swg, sidx, didx, semi):
                rb = wbase + swg * KAGG
                pltpu.async_copy(src_hbm.at[pl.ds(rb, KAGG)], sidx, semi)
                pltpu.async_copy(dst_hbm.at[pl.ds(rb, KAGG)], didx, semi)

            def idx_wait(sidx, didx, semi):
                pltpu.make_async_copy(src_hbm.at[pl.ds(0, KAGG)], sidx,
                                      semi).wait()
                pltpu.make_async_copy(dst_hbm.at[pl.ds(0, KAGG)], didx,
                                      semi).wait()

            def gather_fire(sidx, rows, semg):
                return [pltpu.async_copy(g_hbm.at[sidx.at[kk]],
                                         rows.at[pl.ds(kk * WIN, WIN)], semg)
                        for kk in range(KAGG)]

            def scatter_fire(didx, rows, sems):
                for kk in range(KAGG):
                    pltpu.async_copy(rows.at[pl.ds(kk * WIN, WIN)],
                                     acc.at[didx.at[kk]], sems, add=True)

            def scatter_drain(didx, rows, sems):
                for kk in range(KAGG):
                    pltpu.make_async_copy(rows.at[pl.ds(kk * WIN, WIN)],
                                          acc.at[didx.at[kk]], sems).wait()

            pltpu.sync_copy(g_hbm.at[pl.ds(rbase, rows_per_sub)],
                            acc.at[pl.ds(rbase, rows_per_sub)])
            plsc.subcore_barrier()

            idx_fetch(0, sidxa, didxa, semia)
            idx_fetch(1, sidxb, didxb, semib)

            @pl.loop(0, pairs)
            def _(i):
                @pl.when(i > 0)
                def _():
                    scatter_drain(didxb, rowsb, semsb)
                    idx_fetch(2 * i + 1, sidxb, didxb, semib)

                idx_wait(sidxa, didxa, semia)
                for h in gather_fire(sidxa, rowsa, semga):
                    h.wait()
                scatter_fire(didxa, rowsa, semsa)

                idx_wait(sidxb, didxb, semib)
                hb = gather_fire(sidxb, rowsb, semgb)
                scatter_drain(didxa, rowsa, semsa)

                @pl.when(i < pairs - 1)
                def _():
                    idx_fetch(2 * i + 2, sidxa, didxa, semia)

                for h in hb:
                    h.wait()
                scatter_fire(didxb, rowsb, semsb)

            scatter_drain(didxb, rowsb, semsb)
            plsc.subcore_barrier()
            pltpu.sync_copy(acc.at[pl.ds(rbase, rows_per_sub)],
                            o_hbm.at[pl.ds(rbase, rows_per_sub)])

        @pl.when(c == 0)
        def _():
            run(gl_hbm, ol_hbm)

        @pl.when(c == 1)
        def _():
            run(gr_hbm, or_hbm)

    return k(gl, gr, srcp2, dstp2)


def _agg(g64, srcp2, dstp2, npad, epw):
    gl = lax.slice(g64, (0, 0), (npad, 32))
    gr = lax.slice(g64, (0, 32), (npad, 64))
    ol, orr = _sc_aggregate(gl, gr, srcp2, dstp2, npad, epw)
    return jnp.concatenate([ol, orr], axis=1)


_PREC = lax.Precision.HIGHEST


def _tc_prep(x, dcol, npad):
    grid = -(-npad // RB)

    def body(x_ref, d_ref, g_ref, dinv_ref):
        dinv = lax.rsqrt(d_ref[:, 0:1])
        g_ref[...] = x_ref[...] * dinv
        dinv_ref[...] = jnp.broadcast_to(dinv, (RB, 8))

    return pl.pallas_call(
        body,
        grid=(grid,),
        in_specs=[
            pl.BlockSpec((RB, 64), lambda i: (i, 0)),
            pl.BlockSpec((RB, 8), lambda i: (i, 0)),
        ],
        out_specs=[
            pl.BlockSpec((RB, 64), lambda i: (i, 0)),
            pl.BlockSpec((RB, 8), lambda i: (i, 0)),
        ],
        out_shape=[
            jax.ShapeDtypeStruct((npad, 64), jnp.float32),
            jax.ShapeDtypeStruct((npad, 8), jnp.float32),
        ],
    )(x, dcol)


def _tc_mid(a, dinv, w, b, npad, relu):
    grid = -(-npad // RB)

    def body(a_ref, dinv_ref, w_ref, b_ref, g_ref):
        d = dinv_ref[:, 0:1]
        h = a_ref[...] * d
        h = lax.dot_general(h, w_ref[...], (((1,), (1,)), ((), ())),
                            preferred_element_type=jnp.float32,
                            precision=_PREC) + b_ref[...]
        if relu:
            h = jnp.maximum(h, 0.0)
        g_ref[...] = h * d

    return pl.pallas_call(
        body,
        grid=(grid,),
        in_specs=[
            pl.BlockSpec((RB, 64), lambda i: (i, 0)),
            pl.BlockSpec((RB, 8), lambda i: (i, 0)),
            pl.BlockSpec((64, 64), lambda i: (0, 0)),
            pl.BlockSpec((1, 64), lambda i: (0, 0)),
        ],
        out_specs=pl.BlockSpec((RB, 64), lambda i: (i, 0)),
        out_shape=jax.ShapeDtypeStruct((npad, 64), jnp.float32),
    )(a, dinv, w, b)


def _tc_final(a, dinv, w2, b2, n, out_dim):
    grid = -(-n // RB)

    def body(a_ref, dinv_ref, w_ref, b_ref, o_ref):
        d = dinv_ref[:, 0:1]
        h = a_ref[...] * d
        o = lax.dot_general(h, w_ref[...], (((1,), (1,)), ((), ())),
                            preferred_element_type=jnp.float32,
                            precision=_PREC) + b_ref[...]
        m = jnp.max(o, axis=1, keepdims=True)
        e = o - m
        lse = jnp.log(jnp.sum(jnp.exp(e), axis=1, keepdims=True))
        o_ref[...] = e - lse

    return pl.pallas_call(
        body,
        grid=(grid,),
        in_specs=[
            pl.BlockSpec((RB, 64), lambda i: (i, 0)),
            pl.BlockSpec((RB, 8), lambda i: (i, 0)),
            pl.BlockSpec((out_dim, 64), lambda i: (0, 0)),
            pl.BlockSpec((1, out_dim), lambda i: (0, 0)),
        ],
        out_specs=pl.BlockSpec((RB, out_dim), lambda i: (i, 0)),
        out_shape=jax.ShapeDtypeStruct((n, out_dim), jnp.float32),
    )(a, dinv, w2, b2)


def kernel(x, edge_index, W1, b1, W2, b2):
    n, in_dim = x.shape
    e = edge_index.shape[1]
    hid = W1.shape[0]
    out_dim = W2.shape[0]
    assert in_dim == 64 and hid == 64

    npad = _ceil_to(n + 1, 128)
    per_sub = _ceil_to(-(-e // NSUB), 2 * KAGG * WIN)
    epad = per_sub * NSUB

    src = edge_index[0].astype(jnp.int32)
    dst = edge_index[1].astype(jnp.int32)
    pad_idx = jnp.full((epad - e,), n, dtype=jnp.int32)
    srcp2 = jnp.concatenate([src, pad_idx]).reshape(-1, 128)
    dstp2 = jnp.concatenate([dst, pad_idx]).reshape(-1, 128)

    b1r = b1.reshape(1, hid)
    b2r = b2.reshape(1, out_dim)
    ones32 = jnp.ones((npad, 32), jnp.float32)

    d0l, _ = _sc_aggregate(ones32, ones32, srcp2, dstp2, npad, per_sub)
    dcol = lax.slice(d0l, (0, 0), (npad, 8))
    g1, dinv = _tc_prep(x, dcol, npad)

    a1 = _agg(g1, srcp2, dstp2, npad, per_sub)
    g2 = _tc_mid(a1, dinv, W1, b1r, npad, relu=False)

    a2 = _agg(g2, srcp2, dstp2, npad, per_sub)
    g3 = _tc_mid(a2, dinv, W1, b1r, npad, relu=True)

    a3 = _agg(g3, srcp2, dstp2, npad, per_sub)
    return _tc_final(a3, dinv, W2, b2r, n, out_dim)


# speedup vs baseline: 1.3755x; 1.3755x over previous
import functools

import jax
import jax.numpy as jnp
from jax import lax
from jax.experimental import pallas as pl
from jax.experimental.pallas import tpu as pltpu
from jax.experimental.pallas import tpu_sc as plsc

WIN = 128
KAGG = 2
NSUB = 16
NWCH = 8
RB = 8192


def _ceil_to(x, m):
    return (x + m - 1) // m * m


_SC_PARAMS = pltpu.CompilerParams(use_tc_tiling_on_sc=False)


def _sc_aggregate(zeros32, gflat, srca, srcb, dstp2, npad, epw):
    rows_per_sub = npad // NSUB
    wch = rows_per_sub // NWCH
    wrows = epw // WIN
    pairs = wrows // (2 * KAGG)
    SW = KAGG * WIN

    @functools.partial(
        pl.kernel,
        out_type=jax.ShapeDtypeStruct((npad, 128), jnp.float32),
        mesh=plsc.VectorSubcoreMesh(core_axis_name="c", subcore_axis_name="s"),
        scratch_types=[
            pltpu.VMEM((KAGG, WIN), jnp.int32),
            pltpu.VMEM((KAGG, WIN), jnp.int32),
            pltpu.VMEM((KAGG, WIN), jnp.int32),
            pltpu.VMEM((KAGG, WIN), jnp.int32),
            pltpu.VMEM((SW, 32), jnp.float32),
            pltpu.VMEM((SW, 32), jnp.float32),
            pltpu.VMEM_SHARED((npad, 32), jnp.float32),
            pltpu.SemaphoreType.DMA,
            pltpu.SemaphoreType.DMA,
            pltpu.SemaphoreType.DMA,
            pltpu.SemaphoreType.DMA,
            pltpu.SemaphoreType.DMA,
            pltpu.SemaphoreType.DMA,
        ],
        compiler_params=_SC_PARAMS,
    )
    def k(z_hbm, g_hbm, srca_hbm, srcb_hbm, dst_hbm, o_hbm,
          sidxa, didxa, sidxb, didxb, rowsa, rowsb, acc,
          semia, semib, semga, semgb, semsa, semsb):
        c = lax.axis_index("c")
        s = lax.axis_index("s")
        coff = pl.multiple_of(c * 32, 32)
        rbase = pl.multiple_of(s * rows_per_sub, 8)
        wbase = pl.multiple_of(s * wrows, 8)

        def run(src_hbm):
            def idx_fetch(swg, sidx, didx, semi):
                rb = wbase + swg * KAGG
                pltpu.async_copy(src_hbm.at[pl.ds(rb, KAGG)], sidx, semi)
                pltpu.async_copy(dst_hbm.at[pl.ds(rb, KAGG)], didx, semi)

            def idx_wait(sidx, didx, semi):
                pltpu.make_async_copy(src_hbm.at[pl.ds(0, KAGG)], sidx,
                                      semi).wait()
                pltpu.make_async_copy(dst_hbm.at[pl.ds(0, KAGG)], didx,
                                      semi).wait()

            def gather_fire(sidx, rows, semg):
                return [pltpu.async_copy(g_hbm.at[sidx.at[kk]],
                                         rows.at[pl.ds(kk * WIN, WIN)], semg)
                        for kk in range(KAGG)]

            def scatter_fire(didx, rows, sems):
                for kk in range(KAGG):
                    pltpu.async_copy(rows.at[pl.ds(kk * WIN, WIN)],
                                     acc.at[didx.at[kk]], sems, add=True)

            def scatter_drain(didx, rows, sems):
                for kk in range(KAGG):
                    pltpu.make_async_copy(rows.at[pl.ds(kk * WIN, WIN)],
                                          acc.at[didx.at[kk]], sems).wait()

            pltpu.sync_copy(z_hbm.at[pl.ds(rbase, rows_per_sub)],
                            acc.at[pl.ds(rbase, rows_per_sub)])
            plsc.subcore_barrier()

            idx_fetch(0, sidxa, didxa, semia)
            idx_fetch(1, sidxb, didxb, semib)

            @pl.loop(0, pairs)
            def _(i):
                @pl.when(i > 0)
                def _():
                    scatter_drain(didxb, rowsb, semsb)
                    idx_fetch(2 * i + 1, sidxb, didxb, semib)

                idx_wait(sidxa, didxa, semia)
                for h in gather_fire(sidxa, rowsa, semga):
                    h.wait()
                scatter_fire(didxa, rowsa, semsa)

                idx_wait(sidxb, didxb, semib)
                hb = gather_fire(sidxb, rowsb, semgb)
                scatter_drain(didxa, rowsa, semsa)

                @pl.when(i < pairs - 1)
                def _():
                    idx_fetch(2 * i + 2, sidxa, didxa, semia)

                for h in hb:
                    h.wait()
                scatter_fire(didxb, rowsb, semsb)

            scatter_drain(didxb, rowsb, semsb)
            plsc.subcore_barrier()
            outs = [pltpu.async_copy(
                acc.at[pl.ds(rbase + j * wch, wch)],
                o_hbm.at[pl.ds(rbase + j * wch, wch), pl.ds(coff, 32)],
                semia) for j in range(NWCH)]
            for h in outs:
                h.wait()

        @pl.when(c == 0)
        def _():
            run(srca_hbm)

        @pl.when(c == 1)
        def _():
            run(srcb_hbm)

    return k(zeros32, gflat, srca, srcb, dstp2)


_PREC = lax.Precision.HIGHEST
_Z64 = None


def _tc_prep(x, d0, npad):
    grid = -(-npad // RB)

    def body(x_ref, d_ref, g_ref, dinv_ref):
        dinv = lax.rsqrt(d_ref[:, 0:1] + 1.0)
        g = x_ref[...] * dinv
        g_ref[...] = jnp.concatenate([g, jnp.zeros((RB, 64), jnp.float32)], 1)
        dinv_ref[...] = jnp.broadcast_to(dinv, (RB, 8))

    return pl.pallas_call(
        body,
        grid=(grid,),
        in_specs=[
            pl.BlockSpec((RB, 64), lambda i: (i, 0)),
            pl.BlockSpec((RB, 128), lambda i: (i, 0)),
        ],
        out_specs=[
            pl.BlockSpec((RB, 128), lambda i: (i, 0)),
            pl.BlockSpec((RB, 8), lambda i: (i, 0)),
        ],
        out_shape=[
            jax.ShapeDtypeStruct((npad, 128), jnp.float32),
            jax.ShapeDtypeStruct((npad, 8), jnp.float32),
        ],
    )(x, d0)


def _tc_mid(a, gp, dinv, w, b, npad, relu):
    grid = -(-npad // RB)

    def body(a_ref, gp_ref, dinv_ref, w_ref, b_ref, g_ref):
        d = dinv_ref[:, 0:1]
        h = (a_ref[:, :64] + gp_ref[:, :64]) * d
        h = lax.dot_general(h, w_ref[...], (((1,), (1,)), ((), ())),
                            preferred_element_type=jnp.float32,
                            precision=_PREC) + b_ref[...]
        if relu:
            h = jnp.maximum(h, 0.0)
        g = h * d
        g_ref[...] = jnp.concatenate([g, jnp.zeros((RB, 64), jnp.float32)], 1)

    return pl.pallas_call(
        body,
        grid=(grid,),
        in_specs=[
            pl.BlockSpec((RB, 128), lambda i: (i, 0)),
            pl.BlockSpec((RB, 128), lambda i: (i, 0)),
            pl.BlockSpec((RB, 8), lambda i: (i, 0)),
            pl.BlockSpec((64, 64), lambda i: (0, 0)),
            pl.BlockSpec((1, 64), lambda i: (0, 0)),
        ],
        out_specs=pl.BlockSpec((RB, 128), lambda i: (i, 0)),
        out_shape=jax.ShapeDtypeStruct((npad, 128), jnp.float32),
    )(a, gp, dinv, w, b)


def _tc_final(a, gp, dinv, w2, b2, n, out_dim):
    grid = -(-n // RB)

    def body(a_ref, gp_ref, dinv_ref, w_ref, b_ref, o_ref):
        d = dinv_ref[:, 0:1]
        h = (a_ref[:, :64] + gp_ref[:, :64]) * d
        o = lax.dot_general(h, w_ref[...], (((1,), (1,)), ((), ())),
                            preferred_element_type=jnp.float32,
                            precision=_PREC) + b_ref[...]
        m = jnp.max(o, axis=1, keepdims=True)
        e = o - m
        lse = jnp.log(jnp.sum(jnp.exp(e), axis=1, keepdims=True))
        o_ref[...] = e - lse

    return pl.pallas_call(
        body,
        grid=(grid,),
        in_specs=[
            pl.BlockSpec((RB, 128), lambda i: (i, 0)),
            pl.BlockSpec((RB, 128), lambda i: (i, 0)),
            pl.BlockSpec((RB, 8), lambda i: (i, 0)),
            pl.BlockSpec((out_dim, 64), lambda i: (0, 0)),
            pl.BlockSpec((1, out_dim), lambda i: (0, 0)),
        ],
        out_specs=pl.BlockSpec((RB, out_dim), lambda i: (i, 0)),
        out_shape=jax.ShapeDtypeStruct((n, out_dim), jnp.float32),
    )(a, gp, dinv, w2, b2)


def kernel(x, edge_index, W1, b1, W2, b2):
    n, in_dim = x.shape
    e = edge_index.shape[1]
    hid = W1.shape[0]
    out_dim = W2.shape[0]
    assert in_dim == 64 and hid == 64

    npad = _ceil_to(n + 1, 128)
    per_sub = _ceil_to(-(-e // NSUB), 2 * KAGG * WIN)
    epad = per_sub * NSUB

    src = edge_index[0].astype(jnp.int32)
    dst = edge_index[1].astype(jnp.int32)
    pad_idx = jnp.full((epad - e,), n, dtype=jnp.int32)
    src2 = jnp.concatenate([src, pad_idx]).reshape(-1, 128) * 2
    srca = src2
    srcb = src2 + 1
    dstp2 = jnp.concatenate([dst, pad_idx]).reshape(-1, 128)

    b1r = b1.reshape(1, hid)
    b2r = b2.reshape(1, out_dim)
    zeros32 = jnp.zeros((npad, 32), jnp.float32)
    onesf = jnp.ones((npad * 2, 32), jnp.float32)

    def flat(g):
        return g[:, :64].reshape(npad * 2, 32)

    d0 = _sc_aggregate(zeros32, onesf, srca, srcb, dstp2, npad, per_sub)
    g1, dinv = _tc_prep(x, d0, npad)

    a1 = _sc_aggregate(zeros32, flat(g1), srca, srcb, dstp2, npad, per_sub)
    g2 = _tc_mid(a1, g1, dinv, W1, b1r, npad, relu=False)

    a2 = _sc_aggregate(zeros32, flat(g2), srca, srcb, dstp2, npad, per_sub)
    g3 = _tc_mid(a2, g2, dinv, W1, b1r, npad, relu=True)

    a3 = _sc_aggregate(zeros32, flat(g3), srca, srcb, dstp2, npad, per_sub)
    return _tc_final(a3, g3, dinv, W2, b2r, n, out_dim)
